# baseline (speedup 1.0000x reference)
import jax
from jax.experimental import pallas as pl
from jax.experimental.pallas import tpu as pltpu

_CHUNK = 2048
_SPLIT = 256
_NBUF = 6
_LAG = 3


def kernel(batch, index):
    del index
    total, d = batch.shape
    n = total // _CHUNK

    blocks = []
    for c in range(n):
        if c == 0 or c == n - 1:
            blocks += [(c, off, _SPLIT) for off in range(0, _CHUNK, _SPLIT)]
        else:
            blocks.append((c, 0, _CHUNK))
    nblk = len(blocks)

    def body(in_hbm, *args):
        outs = args[:n]
        buf, in_sem, out_sem = args[n], args[n + 1], args[n + 2]

        def in_copy(k):
            c, off, rows = blocks[k]
            return pltpu.make_async_copy(
                in_hbm.at[pl.ds(c * _CHUNK + off, rows)],
                buf.at[k % _NBUF, pl.ds(0, rows)],
                in_sem.at[k % _NBUF],
            )

        def out_copy(k):
            c, off, rows = blocks[k]
            return pltpu.make_async_copy(
                buf.at[k % _NBUF, pl.ds(0, rows)],
                outs[c].at[pl.ds(off, rows)],
                out_sem.at[k % _NBUF],
            )

        for k in range(_NBUF):
            in_copy(k).start()

        out_waited = [False] * nblk
        for k in range(nblk):
            in_copy(k).wait()
            out_copy(k).start()
            j = k - _LAG
            if j >= 0 and j + _NBUF < nblk:
                out_copy(j).wait()
                out_waited[j] = True
                in_copy(j + _NBUF).start()
        for k in range(nblk):
            if not out_waited[k]:
                out_copy(k).wait()

    return pl.pallas_call(
        body,
        in_specs=[pl.BlockSpec(memory_space=pl.ANY)],
        out_specs=tuple(pl.BlockSpec(memory_space=pl.ANY) for _ in range(n)),
        out_shape=tuple(
            jax.ShapeDtypeStruct((_CHUNK, d), batch.dtype) for _ in range(n)
        ),
        scratch_shapes=[
            pltpu.VMEM((_NBUF, _CHUNK, d), batch.dtype),
            pltpu.SemaphoreType.DMA((_NBUF,)),
            pltpu.SemaphoreType.DMA((_NBUF,)),
        ],
    )(batch)


# speedup vs baseline: 1.0085x; 1.0085x over previous
import jax
from jax.experimental import pallas as pl
from jax.experimental.pallas import tpu as pltpu

_CHUNK = 2048
_SPLIT = 512
_NBUF = 6
_LAG = 3


def kernel(batch, index):
    del index
    total, d = batch.shape
    n = total // _CHUNK

    blocks = []
    for c in range(n):
        if c < 2 or c >= n - 2:
            blocks += [(c, off, _SPLIT) for off in range(0, _CHUNK, _SPLIT)]
        else:
            blocks.append((c, 0, _CHUNK))
    nblk = len(blocks)

    def body(in_hbm, *args):
        outs = args[:n]
        buf, in_sem, out_sem = args[n], args[n + 1], args[n + 2]

        def in_copy(k):
            c, off, rows = blocks[k]
            return pltpu.make_async_copy(
                in_hbm.at[pl.ds(c * _CHUNK + off, rows)],
                buf.at[k % _NBUF, pl.ds(0, rows)],
                in_sem.at[k % _NBUF],
            )

        def out_copy(k):
            c, off, rows = blocks[k]
            return pltpu.make_async_copy(
                buf.at[k % _NBUF, pl.ds(0, rows)],
                outs[c].at[pl.ds(off, rows)],
                out_sem.at[k % _NBUF],
            )

        for k in range(_NBUF):
            in_copy(k).start()

        out_waited = [False] * nblk
        for k in range(nblk):
            in_copy(k).wait()
            out_copy(k).start()
            j = k - _LAG
            if j >= 0 and j + _NBUF < nblk:
                out_copy(j).wait()
                out_waited[j] = True
                in_copy(j + _NBUF).start()
        for k in range(nblk):
            if not out_waited[k]:
                out_copy(k).wait()

    return pl.pallas_call(
        body,
        in_specs=[pl.BlockSpec(memory_space=pl.ANY)],
        out_specs=tuple(pl.BlockSpec(memory_space=pl.ANY) for _ in range(n)),
        out_shape=tuple(
            jax.ShapeDtypeStruct((_CHUNK, d), batch.dtype) for _ in range(n)
        ),
        scratch_shapes=[
            pltpu.VMEM((_NBUF, _CHUNK, d), batch.dtype),
            pltpu.SemaphoreType.DMA((_NBUF,)),
            pltpu.SemaphoreType.DMA((_NBUF,)),
        ],
    )(batch)


# speedup vs baseline: 1.0132x; 1.0047x over previous
import jax
from jax.experimental import pallas as pl
from jax.experimental.pallas import tpu as pltpu

_CHUNK = 2048
_SPLIT = 512
_NBUF = 6
_LAG = 3


def kernel(batch, index):
    del index
    total, d = batch.shape
    n = total // _CHUNK

    blocks = []
    for c in range(n):
        if c == 0 or c == n - 1:
            blocks += [(c, off, _SPLIT) for off in range(0, _CHUNK, _SPLIT)]
        else:
            blocks.append((c, 0, _CHUNK))
    nblk = len(blocks)

    def body(in_hbm, *args):
        outs = args[:n]
        buf, in_sem, out_sem = args[n], args[n + 1], args[n + 2]

        def in_copy(k):
            c, off, rows = blocks[k]
            return pltpu.make_async_copy(
                in_hbm.at[pl.ds(c * _CHUNK + off, rows)],
                buf.at[k % _NBUF, pl.ds(0, rows)],
                in_sem.at[k % _NBUF],
            )

        def out_copy(k):
            c, off, rows = blocks[k]
            return pltpu.make_async_copy(
                buf.at[k % _NBUF, pl.ds(0, rows)],
                outs[c].at[pl.ds(off, rows)],
                out_sem.at[k % _NBUF],
            )

        for k in range(_NBUF):
            in_copy(k).start()

        out_waited = [False] * nblk
        for k in range(nblk):
            in_copy(k).wait()
            out_copy(k).start()
            j = k - _LAG
            if j >= 0 and j + _NBUF < nblk:
                out_copy(j).wait()
                out_waited[j] = True
                in_copy(j + _NBUF).start()
        for k in range(nblk):
            if not out_waited[k]:
                out_copy(k).wait()

    return pl.pallas_call(
        body,
        in_specs=[pl.BlockSpec(memory_space=pl.ANY)],
        out_specs=tuple(pl.BlockSpec(memory_space=pl.ANY) for _ in range(n)),
        out_shape=tuple(
            jax.ShapeDtypeStruct((_CHUNK, d), batch.dtype) for _ in range(n)
        ),
        scratch_shapes=[
            pltpu.VMEM((_NBUF, _CHUNK, d), batch.dtype),
            pltpu.SemaphoreType.DMA((_NBUF,)),
            pltpu.SemaphoreType.DMA((_NBUF,)),
        ],
    )(batch)


# speedup vs baseline: 1.0171x; 1.0039x over previous
import jax
from jax.experimental import pallas as pl
from jax.experimental.pallas import tpu as pltpu

_CHUNK = 2048
_SPLIT = 512
_NBUF = 6
_LAG = 2


def kernel(batch, index):
    del index
    total, d = batch.shape
    n = total // _CHUNK

    blocks = []
    for c in range(n):
        if c == 0 or c == n - 1:
            blocks += [(c, off, _SPLIT) for off in range(0, _CHUNK, _SPLIT)]
        else:
            blocks.append((c, 0, _CHUNK))
    nblk = len(blocks)

    def body(in_hbm, *args):
        outs = args[:n]
        buf, in_sem, out_sem = args[n], args[n + 1], args[n + 2]

        def in_copy(k):
            c, off, rows = blocks[k]
            return pltpu.make_async_copy(
                in_hbm.at[pl.ds(c * _CHUNK + off, rows)],
                buf.at[k % _NBUF, pl.ds(0, rows)],
                in_sem.at[k % _NBUF],
            )

        def out_copy(k):
            c, off, rows = blocks[k]
            return pltpu.make_async_copy(
                buf.at[k % _NBUF, pl.ds(0, rows)],
                outs[c].at[pl.ds(off, rows)],
                out_sem.at[k % _NBUF],
            )

        for k in range(_NBUF):
            in_copy(k).start()

        out_waited = [False] * nblk
        for k in range(nblk):
            in_copy(k).wait()
            out_copy(k).start()
            j = k - _LAG
            if j >= 0 and j + _NBUF < nblk:
                out_copy(j).wait()
                out_waited[j] = True
                in_copy(j + _NBUF).start()
        for k in range(nblk):
            if not out_waited[k]:
                out_copy(k).wait()

    return pl.pallas_call(
        body,
        in_specs=[pl.BlockSpec(memory_space=pl.ANY)],
        out_specs=tuple(pl.BlockSpec(memory_space=pl.ANY) for _ in range(n)),
        out_shape=tuple(
            jax.ShapeDtypeStruct((_CHUNK, d), batch.dtype) for _ in range(n)
        ),
        scratch_shapes=[
            pltpu.VMEM((_NBUF, _CHUNK, d), batch.dtype),
            pltpu.SemaphoreType.DMA((_NBUF,)),
            pltpu.SemaphoreType.DMA((_NBUF,)),
        ],
    )(batch)
